# baseline (speedup 1.0000x reference)
import functools

import jax
import jax.numpy as jnp
from jax import lax
from jax.experimental import pallas as pl
from jax.experimental.pallas import tpu as pltpu
from jax.experimental.pallas import tpu_sc as plsc

B = 16384
S = 52
K = S - 1
D = 128
E = 1000
MAX_NORM = 1.0
EPS = 1e-7

CHUNK = 128
GROUP = 2
GR = GROUP * CHUNK
BLK = 512


def _renorm_body(w_ref, out_ref):
    w = w_ref[...]
    norms = jnp.sqrt(jnp.sum(w * w, axis=1, keepdims=True))
    scale = jnp.minimum(1.0, MAX_NORM / (norms + EPS))
    out_ref[...] = w * scale


def _renorm_table(w):
    return pl.pallas_call(
        _renorm_body,
        out_shape=jax.ShapeDtypeStruct((E, D), jnp.float32),
    )(w)


def _replicate_body(plane_ref, out_ref):
    out_ref[0] = plane_ref[...]


def _tc_replicate(plane):
    return pl.pallas_call(
        _replicate_body,
        grid=(B // BLK, K),
        in_specs=[pl.BlockSpec((BLK, D), lambda j, k: (j, 0))],
        out_specs=pl.BlockSpec((1, BLK, D), lambda j, k: (k, j, 0)),
        out_shape=jax.ShapeDtypeStruct((K, B, D), jnp.float32),
    )(plane)


def _sc_plane(table, sub_idx, nc, ns):
    nw = nc * ns
    bslice = B // nw
    sub_chunks = bslice // CHUNK
    mesh = plsc.VectorSubcoreMesh(
        core_axis_name="c", subcore_axis_name="s",
        num_cores=nc, num_subcores=ns)

    @functools.partial(
        pl.kernel,
        out_type=jax.ShapeDtypeStruct((B, D), jnp.float32),
        mesh=mesh,
        scratch_types=[
            pltpu.VMEM((sub_chunks, CHUNK), jnp.int32),
            pltpu.VMEM((bslice, D), jnp.float32),
            pltpu.VMEM_SHARED((E, D), jnp.float32),
            pltpu.SemaphoreType.DMA,
        ],
    )
    def k(table_hbm, sub_hbm, plane_out, idxs, pool, table_sp, sem):
        wid = lax.axis_index("s") * nc + lax.axis_index("c")
        b0 = wid * bslice

        @pl.when(lax.axis_index("s") == 0)
        def _():
            pltpu.sync_copy(table_hbm, table_sp)

        plsc.subcore_barrier()
        pltpu.sync_copy(sub_hbm.at[wid], idxs)
        for c in range(sub_chunks):
            pltpu.async_copy(table_sp.at[idxs.at[c]],
                             pool.at[pl.ds(c * CHUNK, CHUNK)], sem)
        for c in range(sub_chunks):
            pltpu.make_async_copy(table_sp.at[idxs.at[c]],
                                  pool.at[pl.ds(c * CHUNK, CHUNK)],
                                  sem).wait()
        pltpu.sync_copy(pool, plane_out.at[pl.ds(b0, bslice)])

    return k(table, sub_idx.reshape(nw, sub_chunks, CHUNK))


def _sc_obj(table, obj_idx, nc, ns):
    nw = nc * ns
    rows = (B * K) // nw
    nchunks = rows // CHUNK
    ngroups = nchunks // GROUP
    mesh = plsc.VectorSubcoreMesh(
        core_axis_name="c", subcore_axis_name="s",
        num_cores=nc, num_subcores=ns)

    @functools.partial(
        pl.kernel,
        out_type=jax.ShapeDtypeStruct((B * K, D), jnp.float32),
        mesh=mesh,
        scratch_types=[
            pltpu.VMEM((nchunks, CHUNK), jnp.int32),
            pltpu.VMEM((2, GR, D), jnp.float32),
            pltpu.VMEM_SHARED((E, D), jnp.float32),
            pltpu.SemaphoreType.DMA,
            pltpu.SemaphoreType.DMA,
            pltpu.SemaphoreType.DMA,
            pltpu.SemaphoreType.DMA,
        ],
    )
    def k(table_hbm, obj_hbm, out, idxs, bufs, table_sp,
          gsem0, gsem1, ssem0, ssem1):
        wid = lax.axis_index("s") * nc + lax.axis_index("c")
        base = wid * rows
        gsem = (gsem0, gsem1)
        ssem = (ssem0, ssem1)

        @pl.when(lax.axis_index("s") == 0)
        def _():
            pltpu.sync_copy(table_hbm, table_sp)

        plsc.subcore_barrier()
        pltpu.sync_copy(obj_hbm.at[wid], idxs)
        for c in range(GROUP):
            pltpu.async_copy(
                table_sp.at[idxs.at[c]],
                bufs.at[0].at[pl.ds(c * CHUNK, CHUNK)], gsem[0])

        @pl.loop(0, ngroups, step=2)
        def _(tt):
            for p in range(2):
                t = tt + p
                q = 1 - p
                for c in range(GROUP):
                    pltpu.make_async_copy(
                        table_sp.at[idxs.at[t * GROUP + c]],
                        bufs.at[p].at[pl.ds(c * CHUNK, CHUNK)],
                        gsem[p]).wait()
                pltpu.async_copy(
                    bufs.at[p], out.at[pl.ds(base + t * GR, GR)], ssem[p])
                @pl.when(t > 0)
                def _():
                    pltpu.make_async_copy(
                        bufs.at[q],
                        out.at[pl.ds(base + (t - 1) * GR, GR)],
                        ssem[q]).wait()

                @pl.when(t + 1 < ngroups)
                def _():
                    for c in range(GROUP):
                        pltpu.async_copy(
                            table_sp.at[idxs.at[(t + 1) * GROUP + c]],
                            bufs.at[q].at[pl.ds(c * CHUNK, CHUNK)],
                            gsem[q])

        pltpu.make_async_copy(
            bufs.at[1], out.at[pl.ds(base + (ngroups - 1) * GR, GR)],
            ssem[1]).wait()

    return k(table, obj_idx.reshape(nw, nchunks, CHUNK))


def kernel(inputs, embed_weight):
    scaled = _renorm_table(embed_weight)
    sub_idx = inputs[:, 0]
    obj_idx = inputs[:, 1:].T
    info = plsc.get_sparse_core_info()
    plane = _sc_plane(scaled, sub_idx, info.num_cores, info.num_subcores)
    obj = _sc_obj(scaled, obj_idx, info.num_cores, info.num_subcores)
    sub = _tc_replicate(plane)
    return (sub.transpose(1, 0, 2),
            obj.reshape(K, B, D).transpose(1, 0, 2))


# speedup vs baseline: 2.4377x; 2.4377x over previous
import functools

import jax
import jax.numpy as jnp
from jax import lax
from jax.experimental import pallas as pl
from jax.experimental.pallas import tpu as pltpu
from jax.experimental.pallas import tpu_sc as plsc

B = 16384
S = 52
K = S - 1
D = 128
E = 1000
MAX_NORM = 1.0
EPS = 1e-7

CHUNK = 128
GROUP = 2
GR = GROUP * CHUNK
BLK = 2048


def _renorm_body(w_ref, out_ref):
    w = w_ref[...]
    norms = jnp.sqrt(jnp.sum(w * w, axis=1, keepdims=True))
    scale = jnp.minimum(1.0, MAX_NORM / (norms + EPS))
    out_ref[...] = w * scale


def _renorm_table(w):
    return pl.pallas_call(
        _renorm_body,
        out_shape=jax.ShapeDtypeStruct((E, D), jnp.float32),
    )(w)


def _replicate_body(plane_hbm, out_hbm, vbuf, sem_in, sem_out):
    pltpu.async_copy(plane_hbm, vbuf, sem_in).wait()
    for kk in range(K):
        pltpu.async_copy(vbuf, out_hbm.at[kk], sem_out)
    for kk in range(K):
        pltpu.make_async_copy(vbuf, out_hbm.at[kk], sem_out).wait()


def _tc_replicate(plane):
    return pl.pallas_call(
        _replicate_body,
        in_specs=[pl.BlockSpec(memory_space=pl.ANY)],
        out_specs=pl.BlockSpec(memory_space=pl.ANY),
        out_shape=jax.ShapeDtypeStruct((K, B, D), jnp.float32),
        scratch_shapes=[
            pltpu.VMEM((B, D), jnp.float32),
            pltpu.SemaphoreType.DMA,
            pltpu.SemaphoreType.DMA,
        ],
    )(plane)


def _sc_plane(table, sub_idx, nc, ns):
    nw = nc * ns
    bslice = B // nw
    sub_chunks = bslice // CHUNK
    mesh = plsc.VectorSubcoreMesh(
        core_axis_name="c", subcore_axis_name="s",
        num_cores=nc, num_subcores=ns)

    @functools.partial(
        pl.kernel,
        out_type=jax.ShapeDtypeStruct((B, D), jnp.float32),
        mesh=mesh,
        scratch_types=[
            pltpu.VMEM((sub_chunks, CHUNK), jnp.int32),
            pltpu.VMEM((bslice, D), jnp.float32),
            pltpu.VMEM_SHARED((E, D), jnp.float32),
            pltpu.SemaphoreType.DMA,
        ],
    )
    def k(table_hbm, sub_hbm, plane_out, idxs, pool, table_sp, sem):
        wid = lax.axis_index("s") * nc + lax.axis_index("c")
        b0 = wid * bslice

        @pl.when(lax.axis_index("s") == 0)
        def _():
            pltpu.sync_copy(table_hbm, table_sp)

        plsc.subcore_barrier()
        pltpu.sync_copy(sub_hbm.at[wid], idxs)
        for c in range(sub_chunks):
            pltpu.async_copy(table_sp.at[idxs.at[c]],
                             pool.at[pl.ds(c * CHUNK, CHUNK)], sem)
        for c in range(sub_chunks):
            pltpu.make_async_copy(table_sp.at[idxs.at[c]],
                                  pool.at[pl.ds(c * CHUNK, CHUNK)],
                                  sem).wait()
        pltpu.sync_copy(pool, plane_out.at[pl.ds(b0, bslice)])

    return k(table, sub_idx.reshape(nw, sub_chunks, CHUNK))


def _sc_obj(table, obj_idx, nc, ns):
    nw = nc * ns
    rows = (B * K) // nw
    nchunks = rows // CHUNK
    ngroups = nchunks // GROUP
    mesh = plsc.VectorSubcoreMesh(
        core_axis_name="c", subcore_axis_name="s",
        num_cores=nc, num_subcores=ns)

    @functools.partial(
        pl.kernel,
        out_type=jax.ShapeDtypeStruct((B * K, D), jnp.float32),
        mesh=mesh,
        scratch_types=[
            pltpu.VMEM((nchunks, CHUNK), jnp.int32),
            pltpu.VMEM((2, GR, D), jnp.float32),
            pltpu.VMEM_SHARED((E, D), jnp.float32),
            pltpu.SemaphoreType.DMA,
            pltpu.SemaphoreType.DMA,
            pltpu.SemaphoreType.DMA,
            pltpu.SemaphoreType.DMA,
        ],
    )
    def k(table_hbm, obj_hbm, out, idxs, bufs, table_sp,
          gsem0, gsem1, ssem0, ssem1):
        wid = lax.axis_index("s") * nc + lax.axis_index("c")
        base = wid * rows
        gsem = (gsem0, gsem1)
        ssem = (ssem0, ssem1)

        @pl.when(lax.axis_index("s") == 0)
        def _():
            pltpu.sync_copy(table_hbm, table_sp)

        plsc.subcore_barrier()
        pltpu.sync_copy(obj_hbm.at[wid], idxs)
        for c in range(GROUP):
            pltpu.async_copy(
                table_sp.at[idxs.at[c]],
                bufs.at[0].at[pl.ds(c * CHUNK, CHUNK)], gsem[0])

        @pl.loop(0, ngroups, step=2)
        def _(tt):
            for p in range(2):
                t = tt + p
                q = 1 - p
                for c in range(GROUP):
                    pltpu.make_async_copy(
                        table_sp.at[idxs.at[t * GROUP + c]],
                        bufs.at[p].at[pl.ds(c * CHUNK, CHUNK)],
                        gsem[p]).wait()
                pltpu.async_copy(
                    bufs.at[p], out.at[pl.ds(base + t * GR, GR)], ssem[p])
                @pl.when(t > 0)
                def _():
                    pltpu.make_async_copy(
                        bufs.at[q],
                        out.at[pl.ds(base + (t - 1) * GR, GR)],
                        ssem[q]).wait()

                @pl.when(t + 1 < ngroups)
                def _():
                    for c in range(GROUP):
                        pltpu.async_copy(
                            table_sp.at[idxs.at[(t + 1) * GROUP + c]],
                            bufs.at[q].at[pl.ds(c * CHUNK, CHUNK)],
                            gsem[q])

        pltpu.make_async_copy(
            bufs.at[1], out.at[pl.ds(base + (ngroups - 1) * GR, GR)],
            ssem[1]).wait()

    return k(table, obj_idx.reshape(nw, nchunks, CHUNK))


def kernel(inputs, embed_weight):
    scaled = _renorm_table(embed_weight)
    sub_idx = inputs[:, 0]
    obj_idx = inputs[:, 1:].T
    info = plsc.get_sparse_core_info()
    plane = _sc_plane(scaled, sub_idx, info.num_cores, info.num_subcores)
    obj = _sc_obj(scaled, obj_idx, info.num_cores, info.num_subcores)
    sub = _tc_replicate(plane)
    return (sub.transpose(1, 0, 2),
            obj.reshape(K, B, D).transpose(1, 0, 2))
